# baseline (speedup 1.0000x reference)
import functools

import jax
import jax.numpy as jnp
from jax import lax
from jax.experimental import pallas as pl
from jax.experimental.pallas import tpu as pltpu
from jax.experimental.pallas import tpu_sc as plsc

_NC = 2
_NS = 16
_LANES = 16


def _mesh():
    return plsc.VectorSubcoreMesh(
        core_axis_name="c", subcore_axis_name="s",
        num_cores=_NC, num_subcores=_NS)


_LPAD = 24


def _build_compact(F, B, L):
    NW = _NC * _NS
    NBLK = B // 128

    @functools.partial(
        pl.kernel,
        out_type=jax.ShapeDtypeStruct((F, NBLK, _LPAD, 128), jnp.int32),
        mesh=_mesh(),
        compiler_params=pltpu.CompilerParams(use_tc_tiling_on_sc=True),
        scratch_types=[
            pltpu.VMEM((_LPAD, 128), jnp.int32),
            pltpu.VMEM((_LPAD, 128), jnp.int32),
            pltpu.SemaphoreType.DMA,
        ],
    )
    def compact(idx_hbm, out_hbm, stage0, stage1, sem):
        wid = lax.axis_index("s") * _NC + lax.axis_index("c")
        stages = [stage0, stage1]
        pending = [None, None]
        for f in range(F):
            stage = stages[f % 2]
            if pending[f % 2] is not None:
                pending[f % 2].wait()
                pending[f % 2] = None
            pltpu.sync_copy(
                idx_hbm.at[f, :, pl.ds(pl.multiple_of(wid * 128, 128), 128)],
                stage.at[pl.ds(0, L), :])
            pending[f % 2] = pltpu.async_copy(
                stage, out_hbm.at[f, wid], sem)
        for p in pending:
            if p is not None:
                p.wait()

    return compact


def _build_run(F, B, L, V, D):
    NW = _NC * _NS
    NB = B // NW

    @functools.partial(
        pl.kernel,
        out_type=jax.ShapeDtypeStruct((B, F * D), jnp.float32),
        mesh=_mesh(),
        compiler_params=pltpu.CompilerParams(use_tc_tiling_on_sc=False),
        scratch_types=[
            pltpu.VMEM((L, NB), jnp.int32),
            pltpu.VMEM((L * NB, D), jnp.float32),
            pltpu.VMEM((NB, D), jnp.float32),
            pltpu.VMEM((F, L, _LANES), jnp.float32),
            pltpu.SemaphoreType.DMA,
        ],
    )
    def run(idx_hbm, table_hbm, pwe_hbm, out_hbm, idx_v, rows_v, out_v, pw_v,
            sem):
        wid = lax.axis_index("s") * _NC + lax.axis_index("c")
        pltpu.sync_copy(pwe_hbm, pw_v)

        def f_body(f, _):
            pltpu.sync_copy(idx_hbm.at[f, wid, pl.ds(0, L), :], idx_v)
            cps = [
                pltpu.async_copy(table_hbm.at[idx_v.at[l]],
                                 rows_v.at[pl.ds(l * NB, NB)], sem)
                for l in range(L)
            ]
            for c in cps:
                c.wait()
            wv = [pw_v[f, l, :] for l in range(L)]

            def bag(i, _):
                acc0 = jnp.zeros((_LANES,), jnp.float32)
                acc1 = jnp.zeros((_LANES,), jnp.float32)
                for l in range(L):
                    acc0 = acc0 + wv[l] * rows_v[l * NB + i, 0:16]
                    acc1 = acc1 + wv[l] * rows_v[l * NB + i, 16:32]
                out_v[i, 0:16] = acc0
                out_v[i, 16:32] = acc1
                return 0

            lax.fori_loop(0, NB, bag, 0)
            pltpu.sync_copy(out_v, out_hbm.at[pl.ds(wid * NB, NB),
                                              pl.ds(f * D, D)])
            return 0

        lax.fori_loop(0, F, f_body, 0)

    return run


def kernel(indices, table, pos_weight):
    F, B, L = indices.shape
    V, D = table.shape
    idx_t = jnp.transpose(indices.astype(jnp.int32), (0, 2, 1))
    pwe = jnp.broadcast_to(
        pos_weight.astype(jnp.float32)[:, :, None], (F, L, _LANES))
    idx_dense = _build_compact(F, B, L)(idx_t)
    return _build_run(F, B, L, V, D)(idx_dense, table, pwe)
